# baseline (speedup 1.0000x reference)
import functools

import jax
import jax.numpy as jnp
from jax import lax
from jax.experimental import pallas as pl
from jax.experimental.pallas import tpu as pltpu
from jax.experimental.pallas import tpu_sc as plsc

_EPS = 1e-08
_L = 16


def _rsqrt(a):
    i = lax.bitcast_convert_type(a, jnp.int32)
    i = jnp.int32(0x5F3759DF) - lax.shift_right_logical(i, 1)
    y = lax.bitcast_convert_type(i, jnp.float32)
    for _ in range(3):
        y = y * (1.5 - 0.5 * a * y * y)
    return y


_GATHER_DNUMS = lax.GatherDimensionNumbers(
    offset_dims=(), collapsed_slice_dims=(0,), start_index_map=(0,))


def _lane_sum(v):
    lanes = lax.iota(jnp.int32, _L)
    for k in (8, 4, 2, 1):
        perm = lax.reshape(jnp.bitwise_xor(lanes, k), (_L, 1))
        v = v + lax.gather(v, perm, _GATHER_DNUMS, slice_sizes=(1,),
                           mode=lax.GatherScatterMode.PROMISE_IN_BOUNDS)
    return v


@functools.partial(jax.jit, static_argnums=(4, 5))
def _sc_embed(x_flat, tok_table, pos_table, scale, T, D):
    info = plsc.get_sparse_core_info()
    NC, NS = info.num_cores, info.num_subcores
    NW = NC * NS
    N = x_flat.shape[0]
    seq_per_w = N // T // NW
    nj = D // _L

    mesh = plsc.VectorSubcoreMesh(core_axis_name="c", subcore_axis_name="s")

    @functools.partial(
        pl.kernel,
        mesh=mesh,
        compiler_params=pltpu.CompilerParams(use_tc_tiling_on_sc=False),
        out_type=jax.ShapeDtypeStruct((N, D), jnp.float32),
        scratch_types=[
            pltpu.VMEM((T,), jnp.int32),
            pltpu.VMEM((T, D), jnp.float32),
            pltpu.VMEM((T, D), jnp.float32),
            pltpu.VMEM((T, D), jnp.float32),
            pltpu.VMEM((D,), jnp.float32),
            pltpu.SemaphoreType.DMA,
        ],
    )
    def k(x_hbm, tok_hbm, pos_hbm, scale_hbm, out_hbm,
          idx_v, rows_v, out_v, pos_v, scale_v, sem):
        wid = lax.axis_index("s") * NC + lax.axis_index("c")
        pltpu.sync_copy(pos_hbm.at[pl.ds(0, T)], pos_v)
        pltpu.sync_copy(scale_hbm, scale_v)
        scale_regs = [scale_v[pl.ds(j * _L, _L)] for j in range(nj)]
        base_w = wid * seq_per_w * T

        def seq_body(c, carry):
            base = base_w + c * T
            pltpu.sync_copy(x_hbm.at[pl.ds(base, T)], idx_v)
            pltpu.async_copy(tok_hbm.at[idx_v], rows_v, sem).wait()

            def row_body(t, carry2):
                vs = []
                acc = None
                for j in range(nj):
                    v = (rows_v[t, pl.ds(j * _L, _L)]
                         + pos_v[t, pl.ds(j * _L, _L)])
                    vs.append(v)
                    sq = v * v
                    acc = sq if acc is None else acc + sq
                ms = _lane_sum(acc) * (1.0 / D) + _EPS
                r = _rsqrt(ms)
                for j in range(nj):
                    out_v[t, pl.ds(j * _L, _L)] = vs[j] * (r * scale_regs[j])
                return carry2

            lax.fori_loop(0, T, row_body, 0)
            pltpu.sync_copy(out_v, out_hbm.at[pl.ds(base, T)])
            return carry

        lax.fori_loop(0, seq_per_w, seq_body, 0)

    return k(x_flat, tok_table, pos_table, scale)


def kernel(x, tok_table, pos_table, scale):
    Bz, Tz = x.shape
    D = tok_table.shape[1]
    out = _sc_embed(x.reshape(Bz * Tz), tok_table, pos_table, scale, Tz, D)
    return out.reshape(Bz, Tz, D)
